# baseline (speedup 1.0000x reference)
import dataclasses
import functools

import jax
import jax.numpy as jnp
from jax import lax
from jax.experimental import pallas as pl
from jax.experimental.pallas import tpu as pltpu
from jax.experimental.pallas import tpu_sc as plsc

ROWS, COLS = 8192, 4096
LANES = 16
NW = 32
RPW = ROWS // NW
GROUPS = RPW // LANES
CW = 1024
CHUNKS = COLS // CW
NBUF = 4
N_TASKS = GROUPS * CHUNKS
PAD_W = CW + 1
UNROLL = 8


def kernel(x):
    mesh = plsc.VectorSubcoreMesh(core_axis_name="core", subcore_axis_name="subcore")
    cp = pltpu.CompilerParams()
    if "needs_layout_passes" in pltpu.CompilerParams.__dataclass_fields__:
        cp = dataclasses.replace(cp, needs_layout_passes=False)

    scratch = [pltpu.VMEM((LANES, PAD_W), jnp.float32) for _ in range(NBUF)]
    scratch += [pltpu.SemaphoreType.DMA for _ in range(2 * NBUF)]

    @functools.partial(
        pl.kernel,
        out_type=jax.ShapeDtypeStruct((ROWS, COLS), jnp.float32),
        mesh=mesh,
        compiler_params=cp,
        scratch_types=scratch,
    )
    def run(x_hbm, o_hbm, *sc):
        bufs, in_sems, out_sems = sc[:NBUF], sc[NBUF:2 * NBUF], sc[2 * NBUF:]
        wid = lax.axis_index("core") * 16 + lax.axis_index("subcore")
        row0w = wid * RPW
        row_iota = lax.iota(jnp.int32, LANES)

        def hbm_slice(ref, t):
            g = t // CHUNKS
            c = t % CHUNKS
            return ref.at[pl.ds(row0w + g * LANES, LANES), pl.ds(c * CW, CW)]

        def start_in(t, b):
            pltpu.async_copy(hbm_slice(x_hbm, t), bufs[b].at[:, pl.ds(0, CW)],
                             in_sems[b])

        def wait_in(t, b):
            pltpu.make_async_copy(hbm_slice(x_hbm, t),
                                  bufs[b].at[:, pl.ds(0, CW)], in_sems[b]).wait()

        def start_out(t, b):
            pltpu.async_copy(bufs[b].at[:, pl.ds(0, CW)], hbm_slice(o_hbm, t),
                             out_sems[b])

        def wait_out(t, b):
            pltpu.make_async_copy(bufs[b].at[:, pl.ds(0, CW)],
                                  hbm_slice(o_hbm, t), out_sems[b]).wait()

        def accumulate(b, acc0):
            buf = bufs[b]

            @plsc.parallel_loop(0, CW, 1, unroll=UNROLL, carry=acc0)
            def acc_loop(i, acc):
                cols = jnp.full((LANES,), 0, jnp.int32) + i
                acc = acc + plsc.load_gather(buf, [row_iota, cols])
                plsc.store_scatter(buf, [row_iota, cols], acc)
                return acc

            return acc_loop

        start_in(0, 0)
        start_in(1, 1)

        @pl.loop(0, GROUPS)
        def _(k):
            acc = jnp.zeros((LANES,), jnp.float32)
            for p in range(CHUNKS):
                t = k * CHUNKS + p
                wait_in(t, p)
                acc = accumulate(p, acc if p else jnp.zeros((LANES,), jnp.float32))
                start_out(t, p)
                b2 = (p + 2) % NBUF

                @pl.when(t >= 2)
                def _():
                    wait_out(t - 2, b2)

                @pl.when(t + 2 < N_TASKS)
                def _():
                    start_in(t + 2, b2)

        wait_out(N_TASKS - 2, (N_TASKS - 2) % NBUF)
        wait_out(N_TASKS - 1, (N_TASKS - 1) % NBUF)

    return run(x)


# speedup vs baseline: 4.0646x; 4.0646x over previous
import dataclasses
import functools

import jax
import jax.numpy as jnp
from jax import lax
from jax.experimental import pallas as pl
from jax.experimental.pallas import tpu as pltpu
from jax.experimental.pallas import tpu_sc as plsc

ROWS, COLS = 8192, 4096
LANES = 16
R_BLK = 4
VPR = COLS // LANES
UNROLL = 8


_GATHER_DNUMS = lax.GatherDimensionNumbers(
    offset_dims=(), collapsed_slice_dims=(0,), start_index_map=(0,)
)


def _bcast_last(s):
    idx = jnp.full((LANES, 1), LANES - 1, jnp.int32)
    return lax.gather(
        s, idx, _GATHER_DNUMS, slice_sizes=(1,),
        mode=lax.GatherScatterMode.PROMISE_IN_BOUNDS,
    )


def _scan_block(in_vmem, out_vmem):
    zero = jnp.float32(0.0)

    @plsc.parallel_loop(0, VPR, 1, unroll=UNROLL, carry=(zero,) * R_BLK)
    def _(j, carries):
        carries = list(carries)
        off = j * LANES
        for r in range(R_BLK):
            v = in_vmem[r, pl.ds(off, LANES)]
            s = jnp.cumsum(v) + carries[r]
            out_vmem[r, pl.ds(off, LANES)] = s
            carries[r] = s[LANES - 1]
        return tuple(carries)


def kernel(x):
    mesh = plsc.VectorSubcoreMesh(core_axis_name="core", subcore_axis_name="subcore")
    cp = pltpu.CompilerParams()
    if "needs_layout_passes" in pltpu.CompilerParams.__dataclass_fields__:
        cp = dataclasses.replace(cp, needs_layout_passes=False)

    @functools.partial(
        pl.kernel,
        out_type=jax.ShapeDtypeStruct((ROWS, COLS), jnp.float32),
        mesh=mesh,
        compiler_params=cp,
    )
    def run(x_hbm, o_hbm):
        def body(in_vmem, out_vmem):
            _scan_block(in_vmem, out_vmem)

        pltpu.emit_pipeline(
            body,
            grid=(ROWS // R_BLK,),
            in_specs=[pl.BlockSpec((R_BLK, COLS), lambda i: (i, 0))],
            out_specs=[pl.BlockSpec((R_BLK, COLS), lambda i: (i, 0))],
            core_axis_name=("core", "subcore"),
            dimension_semantics=(pltpu.PARALLEL,),
        )(x_hbm, o_hbm)

    return run(x)


# speedup vs baseline: 4.4326x; 1.0905x over previous
import dataclasses
import functools

import jax
import jax.numpy as jnp
from jax import lax
from jax.experimental import pallas as pl
from jax.experimental.pallas import tpu as pltpu
from jax.experimental.pallas import tpu_sc as plsc

ROWS, COLS = 8192, 4096
LANES = 16
R_BLK = 4
VPR = COLS // LANES
UNROLL = 4


_GATHER_DNUMS = lax.GatherDimensionNumbers(
    offset_dims=(), collapsed_slice_dims=(0,), start_index_map=(0,)
)


def _bcast_last(s):
    idx = jnp.full((LANES, 1), LANES - 1, jnp.int32)
    return lax.gather(
        s, idx, _GATHER_DNUMS, slice_sizes=(1,),
        mode=lax.GatherScatterMode.PROMISE_IN_BOUNDS,
    )


def _scan_block(in_vmem, out_vmem):
    zero = jnp.float32(0.0)

    @plsc.parallel_loop(0, VPR, 1, unroll=UNROLL, carry=(zero,) * R_BLK)
    def _(j, carries):
        carries = list(carries)
        off = j * LANES
        for r in range(R_BLK):
            v = in_vmem[r, pl.ds(off, LANES)]
            s = jnp.cumsum(v) + carries[r]
            out_vmem[r, pl.ds(off, LANES)] = s
            carries[r] = s[LANES - 1]
        return tuple(carries)


def kernel(x):
    mesh = plsc.VectorSubcoreMesh(core_axis_name="core", subcore_axis_name="subcore")
    cp = pltpu.CompilerParams()
    if "needs_layout_passes" in pltpu.CompilerParams.__dataclass_fields__:
        cp = dataclasses.replace(cp, needs_layout_passes=False)

    @functools.partial(
        pl.kernel,
        out_type=jax.ShapeDtypeStruct((ROWS, COLS), jnp.float32),
        mesh=mesh,
        compiler_params=cp,
    )
    def run(x_hbm, o_hbm):
        def body(in_vmem, out_vmem):
            _scan_block(in_vmem, out_vmem)

        pltpu.emit_pipeline(
            body,
            grid=(ROWS // R_BLK,),
            in_specs=[pl.BlockSpec((R_BLK, COLS), lambda i: (i, 0))],
            out_specs=[pl.BlockSpec((R_BLK, COLS), lambda i: (i, 0))],
            core_axis_name=("core", "subcore"),
            dimension_semantics=(pltpu.PARALLEL,),
        )(x_hbm, o_hbm)

    return run(x)
